# baseline (speedup 1.0000x reference)
import functools

import jax
import jax.numpy as jnp
from jax import lax
from jax.experimental import pallas as pl
from jax.experimental.pallas import tpu as pltpu
from jax.experimental.pallas import tpu_sc as plsc

FEATURE_NUM = 26
VOCAB = 100000
EMBED_DIM = 32
BATCH = 16384

_L = 16
_NC = 2
_NS = 16
_NW = _NC * _NS

_ROWS = BATCH * FEATURE_NUM
_NTILE = FEATURE_NUM * VOCAB // 8
_NQ = FEATURE_NUM * VOCAB // 4

_QUADS = _NTILE // 4
_QPW = _QUADS // _NW
_TB = 48
_P1_BLOCKS = 212

_RPW = _ROWS // _NW
_CH = 64
_NCH = _RPW // _CH
_SS = 13
_NSS = _NCH // _SS
_OROWS = _SS * _CH * EMBED_DIM // 128
_OPW = _RPW * EMBED_DIM // 128


def _p1_body(table_hbm, stg_hbm, vb0, vb1, vp0, vp1, rsem0, rsem1, wsem0, wsem1):
    wid = lax.axis_index("s") * _NC + lax.axis_index("c")
    nq = _QPW + jnp.where(wid < 2, 1, 0)
    baseq = wid * _QPW + lax.min(wid, 2)

    def t0_of(k):
        return 4 * (baseq + lax.min(k * (_TB // 4), nq - (_TB // 4)))

    def fire_read(k, vb, rsem):
        pltpu.make_async_copy(
            table_hbm.at[pl.ds(t0_of(k), _TB)], vb, rsem
        ).start()

    def repack(vb, vp):
        def one_tile(i, _):
            for h in range(8):
                for t in range(2):
                    vp[2 * i + h // 4, pl.ds((h % 4) * EMBED_DIM + t * _L, _L)] = (
                        vb[i, h, pl.ds(t * _L, _L)]
                    )
            return 0

        lax.fori_loop(0, _TB, one_tile, 0)

    vbs = (vb0, vb1)
    vps = (vp0, vp1)
    rsems = (rsem0, rsem1)
    wsems = (wsem0, wsem1)

    fire_read(0, vb0, rsem0)
    fire_read(1, vb1, rsem1)

    def step(kk, _):
        for par in range(2):
            k = 2 * kk + par
            pltpu.make_async_copy(
                table_hbm.at[pl.ds(t0_of(k), _TB)], vbs[par], rsems[par]
            ).wait()

            @pl.when(kk >= 1)
            def _wait_wb():
                pltpu.make_async_copy(
                    vps[par], stg_hbm.at[pl.ds(0, 2 * _TB)], wsems[par]
                ).wait()

            repack(vbs[par], vps[par])

            @pl.when(k + 2 < _P1_BLOCKS)
            def _next_read():
                fire_read(k + 2, vbs[par], rsems[par])

            pltpu.make_async_copy(
                vps[par], stg_hbm.at[pl.ds(2 * t0_of(k), 2 * _TB)], wsems[par]
            ).start()
        return 0

    lax.fori_loop(0, _P1_BLOCKS // 2, step, 0)

    for par in range(2):
        pltpu.make_async_copy(
            vps[par], stg_hbm.at[pl.ds(0, 2 * _TB)], wsems[par]
        ).wait()


def _p2_body(x_hbm, stg_hbm, out_hbm, qv, sv, ga, gb, ob,
             gsem0, gsem1, wsem):
    xv = qv
    wid = lax.axis_index("s") * _NC + lax.axis_index("c")
    xrow0 = wid * (_RPW // 128)
    orow0 = wid * _OPW

    pltpu.sync_copy(x_hbm.at[pl.ds(xrow0, _RPW // 128)], xv)

    lanes = lax.iota(jnp.int32, _L)

    def compute_row(j2, _):
        for t in range(128 // _L):
            r = j2 * 128 + t * _L + lanes
            f = lax.rem(r, FEATURE_NUM)
            flat = xv[j2, pl.ds(t * _L, _L)] + f * VOCAB
            qv[j2, pl.ds(t * _L, _L)] = lax.shift_right_logical(flat, 2)
            sv[j2, pl.ds(t * _L, _L)] = lax.bitwise_and(flat, 3)
        return 0

    lax.fori_loop(0, _RPW // 128, compute_row, 0)

    gbufs = (ga, gb)
    gsems = (gsem0, gsem1)

    def fire(j, gbuf, gsem):
        pltpu.make_async_copy(
            stg_hbm.at[qv.at[lax.div(j, 2), pl.ds(lax.rem(j, 2) * _CH, _CH)]],
            gbuf,
            gsem,
        ).start()

    def extract(j, gbuf):
        k16 = lax.rem(j, _SS) * (_CH // 4)
        for g in range(_CH // _L):
            pvec = g * _L + lanes
            svec = sv[lax.div(j, 2), pl.ds(lax.rem(j, 2) * _CH + g * _L, _L)]
            gbase = svec * EMBED_DIM
            orow = k16 + lax.shift_right_logical(pvec, 2)
            ocol0 = lax.bitwise_and(pvec, 3) * EMBED_DIM
            for c in range(EMBED_DIM):
                cc = jnp.full((_L,), c, jnp.int32)
                v = plsc.load_gather(gbuf, [pvec, gbase + cc])
                plsc.store_scatter(ob, [orow, ocol0 + cc], v)

    fire(0, ga, gsem0)
    fire(1, gb, gsem1)

    def step(j, _):
        kk = lax.rem(j, _SS)

        @pl.when(jnp.logical_and(kk == 0, j >= _SS))
        def _wait_wb():
            pltpu.make_async_copy(
                ob, out_hbm.at[pl.ds(orow0, _OROWS)], wsem
            ).wait()

        for par in range(2):

            @pl.when(lax.rem(j, 2) == par)
            def _do():
                pltpu.make_async_copy(
                    stg_hbm.at[
                        qv.at[lax.div(j, 2), pl.ds(lax.rem(j, 2) * _CH, _CH)]
                    ],
                    gbufs[par],
                    gsems[par],
                ).wait()
                extract(j, gbufs[par])

                @pl.when(j + 2 < _NCH)
                def _next():
                    fire(j + 2, gbufs[par], gsems[par])

        @pl.when(kk == _SS - 1)
        def _wb():
            ss = lax.div(j, _SS)
            pltpu.make_async_copy(
                ob, out_hbm.at[pl.ds(orow0 + ss * _OROWS, _OROWS)], wsem
            ).start()

        return 0

    lax.fori_loop(0, _NCH, step, 0)

    pltpu.make_async_copy(ob, out_hbm.at[pl.ds(orow0, _OROWS)], wsem).wait()


@jax.jit
def _run(x2d, table3):
    p1 = pl.kernel(
        _p1_body,
        mesh=plsc.VectorSubcoreMesh(core_axis_name="c", subcore_axis_name="s"),
        out_type=jax.ShapeDtypeStruct((_NQ, 128), jnp.float32),
        scratch_types=[
            pltpu.VMEM((_TB, 8, EMBED_DIM), jnp.float32),
            pltpu.VMEM((_TB, 8, EMBED_DIM), jnp.float32),
            pltpu.VMEM((2 * _TB, 128), jnp.float32),
            pltpu.VMEM((2 * _TB, 128), jnp.float32),
            pltpu.SemaphoreType.DMA,
            pltpu.SemaphoreType.DMA,
            pltpu.SemaphoreType.DMA,
            pltpu.SemaphoreType.DMA,
        ],
        compiler_params=pltpu.CompilerParams(
            use_tc_tiling_on_sc=True, needs_layout_passes=False
        ),
    )
    stg = p1(table3)

    p2 = pl.kernel(
        _p2_body,
        mesh=plsc.VectorSubcoreMesh(core_axis_name="c", subcore_axis_name="s"),
        out_type=jax.ShapeDtypeStruct((_ROWS * EMBED_DIM // 128, 128), jnp.float32),
        scratch_types=[
            pltpu.VMEM((_RPW // 128, 128), jnp.int32),
            pltpu.VMEM((_RPW // 128, 128), jnp.int32),
            pltpu.VMEM((_CH, 128), jnp.float32),
            pltpu.VMEM((_CH, 128), jnp.float32),
            pltpu.VMEM((_OROWS, 128), jnp.float32),
            pltpu.SemaphoreType.DMA,
            pltpu.SemaphoreType.DMA,
            pltpu.SemaphoreType.DMA,
        ],
        compiler_params=pltpu.CompilerParams(
            use_tc_tiling_on_sc=True, needs_layout_passes=False
        ),
    )
    return p2(x2d, stg)


def kernel(x, tables):
    x2d = x.astype(jnp.int32).reshape(_ROWS // 128, 128)
    table3 = tables.reshape(_NTILE, 8, EMBED_DIM)
    out = _run(x2d, table3)
    return out.reshape(BATCH, FEATURE_NUM, EMBED_DIM)
